# Initial kernel scaffold; baseline (speedup 1.0000x reference)
import functools

import jax
import jax.numpy as jnp
from jax import lax
from jax.experimental import pallas as pl
from jax.experimental.pallas import tpu as pltpu
from jax.experimental.pallas import tpu_sc as plsc

VOCAB = 1000000
SEQ = 200
EMBED = 64
BATCH = 4096
LANES = 16

NUM_CORES = 2
NUM_SUBCORES = 16
NW = NUM_CORES * NUM_SUBCORES
SEQ_PER_W = BATCH // NW
CH_SEQ = 4
CH = CH_SEQ * SEQ
N_CHUNKS = SEQ_PER_W // CH_SEQ


@functools.partial(
    pl.kernel,
    mesh=plsc.VectorSubcoreMesh(core_axis_name="c", subcore_axis_name="s"),
    out_type=jax.ShapeDtypeStruct((BATCH * SEQ, EMBED), jnp.float32),
    scratch_types=[
        pltpu.VMEM((CH,), jnp.int32),
        pltpu.VMEM((CH, EMBED), jnp.float32),
        pltpu.VMEM((SEQ, EMBED), jnp.float32),
        pltpu.SemaphoreType.DMA,
    ],
)
def _embed_lookup(idx_hbm, tok_hbm, pos_hbm, out_hbm, idx_v, rows_v, pos_v, sem):
    wid = lax.axis_index("s") * NUM_CORES + lax.axis_index("c")
    pltpu.sync_copy(pos_hbm, pos_v)
    base0 = wid * (SEQ_PER_W * SEQ)

    def chunk_body(g, carry):
        base = base0 + g * CH
        pltpu.sync_copy(idx_hbm.at[pl.ds(base, CH)], idx_v)
        pltpu.async_copy(tok_hbm.at[idx_v], rows_v, sem).wait()

        def add_body(s, c2):
            for k in range(CH_SEQ):
                for c in range(EMBED // LANES):
                    rows_v[k * SEQ + s, pl.ds(c * LANES, LANES)] += pos_v[
                        s, pl.ds(c * LANES, LANES)
                    ]
            return c2

        lax.fori_loop(0, SEQ, add_body, 0)
        pltpu.sync_copy(rows_v, out_hbm.at[pl.ds(base, CH)])
        return carry

    lax.fori_loop(0, N_CHUNKS, chunk_body, 0)


def kernel(inputs, token_table, position_table):
    idx = inputs.reshape(-1).astype(jnp.int32)
    out = _embed_lookup(idx, token_table, position_table)
    return out.reshape(BATCH, SEQ, EMBED)


# speedup vs baseline: 2.5824x; 2.5824x over previous
import functools

import jax
import jax.numpy as jnp
from jax import lax
from jax.experimental import pallas as pl
from jax.experimental.pallas import tpu as pltpu
from jax.experimental.pallas import tpu_sc as plsc

VOCAB = 1000000
SEQ = 200
EMBED = 64
BATCH = 4096
LANES = 16

NUM_CORES = 2
NUM_SUBCORES = 16
NW = NUM_CORES * NUM_SUBCORES
SEQ_PER_W = BATCH // NW
CH_SEQ = 4
CH = CH_SEQ * SEQ
N_CHUNKS = SEQ_PER_W // CH_SEQ


@functools.partial(
    pl.kernel,
    mesh=plsc.VectorSubcoreMesh(core_axis_name="c", subcore_axis_name="s"),
    out_type=jax.ShapeDtypeStruct((BATCH * SEQ, EMBED), jnp.float32),
    scratch_types=[
        pltpu.VMEM((CH,), jnp.int32),
        pltpu.VMEM((CH, EMBED), jnp.float32),
        pltpu.VMEM((SEQ, EMBED), jnp.float32),
        pltpu.SemaphoreType.DMA,
    ],
    compiler_params=pltpu.CompilerParams(use_tc_tiling_on_sc=False),
)
def _embed_lookup(idx_hbm, tok_hbm, pos_hbm, out_hbm, idx_v, rows_v, pos_v, sem):
    wid = lax.axis_index("s") * NUM_CORES + lax.axis_index("c")
    pltpu.sync_copy(pos_hbm, pos_v)
    base0 = wid * (SEQ_PER_W * SEQ)

    def chunk_body(g, carry):
        base = base0 + g * CH
        pltpu.sync_copy(idx_hbm.at[pl.ds(base, CH)], idx_v)
        pltpu.async_copy(tok_hbm.at[idx_v], rows_v, sem).wait()

        def add_body(s, c2):
            for k in range(CH_SEQ):
                for c in range(EMBED // LANES):
                    rows_v[k * SEQ + s, pl.ds(c * LANES, LANES)] += pos_v[
                        s, pl.ds(c * LANES, LANES)
                    ]
            return c2

        lax.fori_loop(0, SEQ, add_body, 0)
        pltpu.sync_copy(rows_v, out_hbm.at[pl.ds(base, CH)])
        return carry

    lax.fori_loop(0, N_CHUNKS, chunk_body, 0)


def kernel(inputs, token_table, position_table):
    idx = inputs.reshape(-1).astype(jnp.int32)
    out = _embed_lookup(idx, token_table, position_table)
    return out.reshape(BATCH, SEQ, EMBED)


# speedup vs baseline: 4.6856x; 1.8145x over previous
import functools

import jax
import jax.numpy as jnp
from jax import lax
from jax.experimental import pallas as pl
from jax.experimental.pallas import tpu as pltpu
from jax.experimental.pallas import tpu_sc as plsc

VOCAB = 1000000
SEQ = 200
EMBED = 64
BATCH = 4096
LANES = 16

NUM_CORES = 2
NUM_SUBCORES = 16
NW = NUM_CORES * NUM_SUBCORES
BCOL = BATCH // NW
EGROUPS = BCOL // LANES
NBUF = 4
NTR = 2


@functools.partial(
    pl.kernel,
    mesh=plsc.VectorSubcoreMesh(core_axis_name="c", subcore_axis_name="s"),
    out_type=jax.ShapeDtypeStruct(
        (SEQ, EMBED // 8, BATCH // BCOL, 8, BCOL), jnp.float32),
    scratch_types=[
        pltpu.VMEM((SEQ, BCOL), jnp.int32),
        [pltpu.VMEM((BCOL, EMBED), jnp.float32) for _ in range(NBUF)],
        [pltpu.VMEM((EMBED, BCOL), jnp.float32) for _ in range(NTR)],
        pltpu.VMEM((SEQ * EMBED,), jnp.float32),
        [pltpu.SemaphoreType.DMA for _ in range(NBUF)],
        [pltpu.SemaphoreType.DMA for _ in range(NTR)],
    ],
    compiler_params=pltpu.CompilerParams(
        use_tc_tiling_on_sc=False, needs_layout_passes=False),
)
def _embed_lookup(idx_hbm, tok_hbm, pos_hbm, out_hbm,
                  idx_v, rows, trans, pos_v, gsem, wsem):
    wid = lax.axis_index("s") * NUM_CORES + lax.axis_index("c")
    b0 = wid * BCOL

    pltpu.sync_copy(pos_hbm, pos_v)
    pltpu.sync_copy(idx_hbm.at[:, pl.ds(b0, BCOL)], idx_v)

    iota = lax.iota(jnp.int32, LANES)

    def gather(s, r):
        pltpu.async_copy(tok_hbm.at[idx_v.at[s]], rows[r], gsem[r])

    def put(s, t):
        for et in range(EMBED // 8):
            pltpu.async_copy(trans[t].at[pl.ds(et * 8, 8), :],
                             out_hbm.at[s, et, wid], wsem[t])

    def drain(s, t):
        for et in range(EMBED // 8):
            pltpu.make_async_copy(trans[t].at[pl.ds(et * 8, 8), :],
                                  out_hbm.at[s, et, wid], wsem[t]).wait()

    def transpose_add(s, r, t):
        base = s * EMBED

        @plsc.parallel_loop(0, EMBED, unroll=4)
        def e_body(e):
            erot = (iota + e) & (EMBED - 1)
            pvec = plsc.load_gather(pos_v, [erot + base])
            for g in range(EGROUPS):
                bvec = iota + (g * LANES)
                vals = plsc.load_gather(rows[r], [bvec, erot])
                plsc.store_scatter(trans[t], [erot, bvec], vals + pvec)

    def unit(s, r, t):
        pltpu.make_async_copy(tok_hbm.at[idx_v.at[s]], rows[r], gsem[r]).wait()

        @pl.when(s + NBUF - 1 < SEQ)
        def _():
            gather(s + NBUF - 1, (r + NBUF - 1) % NBUF)

        @pl.when(s >= NTR)
        def _():
            drain(s - NTR, t)

        transpose_add(s, r, t)
        put(s, t)

    for r in range(NBUF - 1):
        gather(r, r)

    def quad_body(v, carry):
        s = v * NBUF
        for j in range(NBUF):
            unit(s + j, j, j % NTR)
        return carry

    lax.fori_loop(0, SEQ // NBUF, quad_body, 0)

    drain(SEQ - 2, 0)
    drain(SEQ - 1, 1)


def kernel(inputs, token_table, position_table):
    idx_t = inputs.T
    pos_flat = position_table.reshape(-1)
    out = _embed_lookup(idx_t, token_table, pos_flat)
    return out.transpose(2, 4, 0, 1, 3).reshape(BATCH, SEQ, EMBED)
